# Initial kernel scaffold; baseline (speedup 1.0000x reference)
import functools

import jax
import jax.numpy as jnp
from jax import lax
from jax.experimental import pallas as pl
from jax.experimental.pallas import tpu as pltpu
from jax.experimental.pallas import tpu_sc as plsc

VOCAB = 1000000
EMB = 64
BATCH = 16384
NEG = 20

NC = 2
NS = 16
L = 16
NW = NC * NS
RPW = BATCH // NW
R = 32
NCHUNK = RPW // R
NIDX = R * NEG
NSEG = NIDX // 128


def _sc_scores(center_w, context_w, neg2d, center_emb, context_emb):
    mesh = plsc.VectorSubcoreMesh(core_axis_name="c", subcore_axis_name="s")

    @functools.partial(
        pl.kernel,
        mesh=mesh,
        out_type=jax.ShapeDtypeStruct((NW, NEG + 1, RPW), jnp.float32),
        scratch_types=[
            pltpu.VMEM((R,), jnp.int32),
            pltpu.VMEM((R,), jnp.int32),
            pltpu.VMEM((NSEG, 128), jnp.int32),
            pltpu.VMEM((R, EMB), jnp.float32),
            pltpu.VMEM((R, EMB), jnp.float32),
            pltpu.VMEM((NIDX, EMB), jnp.float32),
            pltpu.VMEM((NEG + 1, RPW), jnp.float32),
            pltpu.SemaphoreType.DMA,
        ],
    )
    def k(cw, xw, nw2, cemb, xemb, out, cidx, xidx, nidx, ubuf, vbuf, nbuf,
          sbuf, sem):
        wid = lax.axis_index("s") * NC + lax.axis_index("c")
        base = wid * RPW
        iota = lax.iota(jnp.int32, L)

        def chunk_body(c, carry):
            rb = base + c * R
            pltpu.sync_copy(cw.at[pl.ds(rb, R)], cidx)
            pltpu.sync_copy(xw.at[pl.ds(rb, R)], xidx)
            pltpu.sync_copy(
                nw2.at[pl.ds(wid * (NCHUNK * NSEG) + c * NSEG, NSEG)], nidx)
            cps = [
                pltpu.async_copy(cemb.at[cidx], ubuf, sem),
                pltpu.async_copy(xemb.at[xidx], vbuf, sem),
            ]
            for j in range(NSEG):
                cps.append(
                    pltpu.async_copy(xemb.at[nidx.at[j]],
                                     nbuf.at[pl.ds(j * 128, 128), :], sem))
            for cp in cps:
                cp.wait()

            for g in range(R // L):
                urow = g * L + iota
                nrow = urow * NEG

                def dbody(d, accs):
                    col = jnp.full((L,), d, jnp.int32)
                    uc = plsc.load_gather(ubuf, [urow, col])
                    vc = plsc.load_gather(vbuf, [urow, col])
                    new = [accs[0] + uc * vc]
                    for kk in range(NEG):
                        ncol = plsc.load_gather(nbuf, [nrow + kk, col])
                        new.append(accs[kk + 1] + uc * ncol)
                    return tuple(new)

                accs = lax.fori_loop(
                    0, EMB, dbody,
                    tuple(jnp.zeros((L,), jnp.float32)
                          for _ in range(NEG + 1)))
                off = c * R + g * L
                sbuf[0, pl.ds(off, L)] = -accs[0]
                for kk in range(NEG):
                    sbuf[kk + 1, pl.ds(off, L)] = accs[kk + 1]
            return carry

        lax.fori_loop(0, NCHUNK, chunk_body, jnp.int32(0))
        pltpu.sync_copy(sbuf, out.at[wid])

    return k(center_w, context_w, neg2d, center_emb, context_emb)


def _tc_loss(scores2d):
    def body(s_ref, o_ref):
        s = s_ref[...]
        sp = jnp.maximum(s, 0.0) + jnp.log1p(jnp.exp(-jnp.abs(s)))
        o_ref[0, 0] = jnp.sum(sp) * (1.0 / BATCH)

    return pl.pallas_call(
        body,
        out_shape=jax.ShapeDtypeStruct((1, 1), jnp.float32),
    )(scores2d)


def kernel(center_w, context_w, negative_ws, center_emb, context_emb):
    cw = center_w.astype(jnp.int32)
    xw = context_w.astype(jnp.int32)
    neg2d = negative_ws.astype(jnp.int32).reshape(BATCH * NEG // 128, 128)
    scores = _sc_scores(cw, xw, neg2d, center_emb, context_emb)
    loss = _tc_loss(scores.reshape(NW * (NEG + 1), RPW))
    return loss[0, 0]


# speedup vs baseline: 4.4665x; 4.4665x over previous
import functools

import jax
import jax.numpy as jnp
from jax import lax
from jax.experimental import pallas as pl
from jax.experimental.pallas import tpu as pltpu
from jax.experimental.pallas import tpu_sc as plsc

VOCAB = 1000000
EMB = 64
BATCH = 16384
NEG = 20
NEGP = 24

NC = 2
NS = 16
L = 16
NW = NC * NS
RPW = BATCH // NW
R = 16
NCHUNK = RPW // R
NIDX = R * NEG


def _sc_scores(cidx, xidx, nidx, cpar, xpar, npar, cembp, xembp):
    mesh = plsc.VectorSubcoreMesh(core_axis_name="c", subcore_axis_name="s")

    @functools.partial(
        pl.kernel,
        mesh=mesh,
        out_type=jax.ShapeDtypeStruct((NW, NEG + 1, RPW), jnp.float32),
        scratch_types=[
            pltpu.VMEM((RPW,), jnp.int32),
            pltpu.VMEM((RPW,), jnp.int32),
            pltpu.VMEM((RPW * NEG,), jnp.int32),
            pltpu.VMEM((RPW,), jnp.float32),
            pltpu.VMEM((RPW,), jnp.float32),
            pltpu.VMEM((RPW * NEGP,), jnp.float32),
            pltpu.VMEM((R, 2 * EMB), jnp.float32),
            pltpu.VMEM((R, 2 * EMB), jnp.float32),
            pltpu.VMEM((NIDX, 2 * EMB), jnp.float32),
            pltpu.VMEM(((NEG + 1) * R, L), jnp.float32),
            pltpu.VMEM((NEG + 1, RPW), jnp.float32),
            pltpu.SemaphoreType.DMA,
        ],
    )
    def k(cih, xih, nih, cph, xph, nph, cemb, xemb, out,
          civ, xiv, niv, cpv, xpv, npv, ubuf, vbuf, nbuf, pbuf, sbuf, sem):
        wid = lax.axis_index("s") * NC + lax.axis_index("c")
        base = wid * RPW
        lane = lax.iota(jnp.int32, L)
        revperm = (((lane & 1) << 3) | ((lane & 2) << 1)
                   | ((lane & 4) >> 1) | ((lane & 8) >> 3))

        pltpu.sync_copy(cih.at[pl.ds(base, RPW)], civ)
        pltpu.sync_copy(xih.at[pl.ds(base, RPW)], xiv)
        pltpu.sync_copy(nih.at[pl.ds(base * NEG, RPW * NEG)], niv)
        pltpu.sync_copy(cph.at[pl.ds(base, RPW)], cpv)
        pltpu.sync_copy(xph.at[pl.ds(base, RPW)], xpv)
        pltpu.sync_copy(nph.at[pl.ds(base * NEGP, RPW * NEGP)], npv)

        def chunk_body(c, carry):
            cps = [
                pltpu.async_copy(cemb.at[civ.at[pl.ds(c * R, R)]], ubuf,
                                 sem),
                pltpu.async_copy(xemb.at[xiv.at[pl.ds(c * R, R)]], vbuf,
                                 sem),
                pltpu.async_copy(
                    xemb.at[niv.at[pl.ds(c * NIDX, 128)]],
                    nbuf.at[pl.ds(0, 128), :], sem),
                pltpu.async_copy(
                    xemb.at[niv.at[pl.ds(c * NIDX + 128, 128)]],
                    nbuf.at[pl.ds(128, 128), :], sem),
                pltpu.async_copy(
                    xemb.at[niv.at[pl.ds(c * NIDX + 256, 64)]],
                    nbuf.at[pl.ds(256, 64), :], sem),
            ]
            for cp in cps:
                cp.wait()

            cp16 = cpv[pl.ds(c * R, L)]
            xp16 = xpv[pl.ds(c * R, L)]
            for r in range(R):
                rsplat = jnp.full((L,), r, jnp.int32)
                pu = cp16[rsplat]
                us = []
                for j in range(4):
                    lo = ubuf[r, pl.ds(j * L, L)]
                    hi = ubuf[r, pl.ds(EMB + j * L, L)]
                    us.append(lo + pu * (hi - lo))
                pv = xp16[rsplat]
                pos = None
                for j in range(4):
                    lo = vbuf[r, pl.ds(j * L, L)]
                    hi = vbuf[r, pl.ds(EMB + j * L, L)]
                    vs = lo + pv * (hi - lo)
                    pos = us[j] * vs if j == 0 else pos + us[j] * vs
                pbuf[r, :] = pos
                npa = npv[pl.ds(c * R * NEGP + r * NEGP, L)]
                npb = npv[pl.ds(c * R * NEGP + r * NEGP + 8, L)]
                for kk in range(NEG):
                    if kk < L:
                        pn = npa[jnp.full((L,), kk, jnp.int32)]
                    else:
                        pn = npb[jnp.full((L,), kk - 8, jnp.int32)]
                    nrow = r * NEG + kk
                    acc = None
                    for j in range(4):
                        lo = nbuf[nrow, pl.ds(j * L, L)]
                        hi = nbuf[nrow, pl.ds(EMB + j * L, L)]
                        ns = lo + pn * (hi - lo)
                        acc = us[j] * ns if j == 0 else acc + us[j] * ns
                    pbuf[(kk + 1) * R + r, :] = acc

            for ks in range(NEG + 1):
                vecs = [pbuf[ks * R + r, :] for r in range(R)]
                h = 8
                while len(vecs) > 1:
                    perm = lane ^ h
                    vecs = [
                        jnp.where((lane & h) == 0, a + a[perm], b + b[perm])
                        for a, b in zip(vecs[0::2], vecs[1::2])
                    ]
                    h //= 2
                s = vecs[0][revperm]
                sbuf[ks, pl.ds(c * R, L)] = -s if ks == 0 else s
            return carry

        lax.fori_loop(0, NCHUNK, chunk_body, jnp.int32(0))
        pltpu.sync_copy(sbuf, out.at[wid])

    return k(cidx, xidx, nidx, cpar, xpar, npar, cembp, xembp)


def _tc_loss(scores2d):
    def body(s_ref, o_ref):
        s = s_ref[...]
        sp = jnp.maximum(s, 0.0) + jnp.log1p(jnp.exp(-jnp.abs(s)))
        o_ref[...] = (jnp.sum(sp) * (1.0 / BATCH)).reshape(1, 1)

    return pl.pallas_call(
        body,
        out_shape=jax.ShapeDtypeStruct((1, 1), jnp.float32),
    )(scores2d)


def kernel(center_w, context_w, negative_ws, center_emb, context_emb):
    cw = center_w.astype(jnp.int32)
    xw = context_w.astype(jnp.int32)
    nw = negative_ws.astype(jnp.int32)
    cidx = cw >> 1
    xidx = xw >> 1
    nidx = (nw >> 1).reshape(BATCH * NEG)
    cpar = (cw & 1).astype(jnp.float32)
    xpar = (xw & 1).astype(jnp.float32)
    npar = jnp.pad((nw & 1).astype(jnp.float32), ((0, 0), (0, NEGP - NEG)))
    npar = npar.reshape(BATCH * NEGP)
    cembp = center_emb.reshape(VOCAB // 2, 2 * EMB)
    xembp = context_emb.reshape(VOCAB // 2, 2 * EMB)
    scores = _sc_scores(cidx, xidx, nidx, cpar, xpar, npar, cembp, xembp)
    loss = _tc_loss(scores.reshape(NW * (NEG + 1), RPW))
    return loss[0, 0]
